# baseline (speedup 1.0000x reference)
import jax
import jax.numpy as jnp
from jax import lax
from jax.experimental import pallas as pl
from jax.experimental.pallas import tpu as pltpu
from jax.experimental.pallas import tpu_sc as plsc

N_OP = 50000
N_M = 2000
N_TM = 8
HID = 64
NL = 3

E_EL_PAD = 802816
E_PR_PAD = 212992
N_ACT_PAD = 802816

N_OP_HALF = 25000
OP_ACC = 25088
M_ACC = 2048
D_ACC = 50048

CH = 128

_MESH = plsc.VectorSubcoreMesh(core_axis_name="c", subcore_axis_name="s", num_cores=2, num_subcores=16)
_SC_PARAMS = pltpu.CompilerParams(use_tc_tiling_on_sc=False)
_SC_PARAMS_NL = pltpu.CompilerParams(use_tc_tiling_on_sc=False,
                                     needs_layout_passes=False)


def _zero2d(buf, rows):
    def body(i, _):
        for j in range(4):
            buf[i, pl.ds(j * 16, 16)] = jnp.zeros((16,), jnp.float32)
        return 0
    lax.fori_loop(0, rows, body, 0)


def _zero1d(buf, n):
    def body(i, _):
        buf[pl.ds(i * 16, 16)] = jnp.zeros((16,), jnp.float32)
        return 0
    lax.fori_loop(0, n // 16, body, 0)


def _addrows(dst_ref, src_ref, n):
    def body(i, _):
        for j in range(4):
            sl = pl.ds(j * 16, 16)
            dst_ref[i, sl] = dst_ref[i, sl] + src_ref[i, sl]
        return 0
    lax.fori_loop(0, n, body, 0)


def _ring_gs(nb, bpc, t0, src_s, src_d, tab, gather_by_s, scat_by_s,
             clamp_base, acc, ibs, ibd, lms, grows, si, sg, ss):
    blk = bpc * CH
    n = nb * bpc

    def issue_i(bb, t):
        pltpu.async_copy(src_s.at[pl.ds(t0 + bb * blk, blk)], ibs[t].at[pl.ds(0, blk)], si)
        pltpu.async_copy(src_d.at[pl.ds(t0 + bb * blk, blk)], ibd[t].at[pl.ds(0, blk)], si)

    def wait_i(bb, t):
        pltpu.make_async_copy(src_s.at[pl.ds(t0 + bb * blk, blk)], ibs[t].at[pl.ds(0, blk)], si).wait()
        pltpu.make_async_copy(src_d.at[pl.ds(t0 + bb * blk, blk)], ibd[t].at[pl.ds(0, blk)], si).wait()

    def issue_g(t, j, cur):
        ib = ibs[t] if gather_by_s else ibd[t]
        pltpu.async_copy(tab.at[ib.at[pl.ds(j * CH, CH)]], grows[cur], sg)

    def wait_g(cur):
        ib = ibs[0] if gather_by_s else ibd[0]
        pltpu.make_async_copy(tab.at[ib.at[pl.ds(0, CH)]], grows[cur], sg).wait()

    def issue_s(prv):
        pltpu.async_copy(grows[prv], acc.at[lms[prv]], ss, add=True)

    def wait_s(prv):
        pltpu.make_async_copy(grows[prv], acc.at[lms[prv]], ss).wait()

    def lm_compute(t, j, cur):
        ib = ibs[t] if scat_by_s else ibd[t]
        for q in range(CH // 16):
            v = ib[pl.ds(j * CH + q * 16, 16)]
            if clamp_base is not None:
                l = v - clamp_base
                ok = (l >= 0) & (l < N_OP_HALF)
                lms[cur][pl.ds(q * 16, 16)] = jnp.where(ok, l, N_OP_HALF)
            else:
                lms[cur][pl.ds(q * 16, 16)] = v

    def pair_body(kk, _):
        for t in range(2):
            bb = 2 * kk + t
            wait_i(bb, t)
            for j in range(bpc):
                k = bb * bpc + j
                cur = j % 2
                prv = 1 - cur
                @pl.when(k >= 2)
                def _():
                    wait_s(cur)
                lm_compute(t, j, cur)
                issue_g(t, j, cur)

                @pl.when(k >= 1)
                def _():
                    wait_g(prv)
                    issue_s(prv)

            @pl.when(bb + 1 < nb)
            def _():
                issue_i(bb + 1, 1 - t)
        return 0

    issue_i(0, 0)
    lax.fori_loop(0, nb // 2, pair_body, 0)
    wait_g(1)
    issue_s(1)
    wait_s(0)
    wait_s(1)


def _deg_body(src, dst, ppd, ones_h, dme, dm, dpp,
              acc_me, acc_m, acc_pp, idx, ones_v, zbuf):
    c = lax.axis_index("c")
    s = lax.axis_index("s")
    _zero1d(zbuf, 3128)
    pltpu.sync_copy(zbuf, acc_me.at[pl.ds(s * 3128, 3128)])
    pltpu.sync_copy(zbuf, acc_pp.at[pl.ds(s * 3128, 3128)])
    pltpu.sync_copy(zbuf.at[pl.ds(0, 128)], acc_m.at[pl.ds(s * 128, 128)])
    pltpu.sync_copy(ones_h, ones_v)
    plsc.subcore_barrier()

    def el_chunk(k, _):
        b = c * 401408 + s * 25088 + k * 512
        pltpu.sync_copy(src.at[pl.ds(b, 512)], idx)
        pltpu.sync_copy(ones_v, acc_me.at[idx], add=True)
        pltpu.sync_copy(dst.at[pl.ds(b, 512)], idx)
        pltpu.sync_copy(ones_v, acc_m.at[idx], add=True)
        return 0
    lax.fori_loop(0, 49, el_chunk, 0)

    def pp_chunk(k, _):
        b = c * 106496 + s * 6656 + k * 512
        pltpu.sync_copy(ppd.at[pl.ds(b, 512)], idx)
        pltpu.sync_copy(ones_v, acc_pp.at[idx], add=True)
        return 0
    lax.fori_loop(0, 13, pp_chunk, 0)
    plsc.subcore_barrier()
    pltpu.sync_copy(acc_me.at[pl.ds(s * 3128, 3128)], zbuf)
    pltpu.sync_copy(zbuf, dme.at[pl.ds(c * D_ACC + s * 3128, 3128)])
    pltpu.sync_copy(acc_pp.at[pl.ds(s * 3128, 3128)], zbuf)
    pltpu.sync_copy(zbuf, dpp.at[pl.ds(c * D_ACC + s * 3128, 3128)])
    pltpu.sync_copy(acc_m.at[pl.ds(s * 128, 128)], zbuf.at[pl.ds(0, 128)])
    pltpu.sync_copy(zbuf.at[pl.ds(0, 128)], dm.at[pl.ds(c * M_ACC + s * 128, 128)])


_deg_call = pl.kernel(
    _deg_body,
    out_type=(
        jax.ShapeDtypeStruct((2 * D_ACC,), jnp.float32),
        jax.ShapeDtypeStruct((2 * M_ACC,), jnp.float32),
        jax.ShapeDtypeStruct((2 * D_ACC,), jnp.float32),
    ),
    mesh=_MESH,
    compiler_params=_SC_PARAMS,
    scratch_types=[
        pltpu.VMEM_SHARED((D_ACC,), jnp.float32),
        pltpu.VMEM_SHARED((M_ACC,), jnp.float32),
        pltpu.VMEM_SHARED((D_ACC,), jnp.float32),
        pltpu.VMEM((512,), jnp.int32),
        pltpu.VMEM((512,), jnp.float32),
        pltpu.VMEM((3128,), jnp.float32),
    ],
)


def _layer_body(xop, bme, src, dst, psrc, pdst, me_out, em_out, pp_out,
                acc_me, acc_em, ibs0, ibs1, ibd0, ibd1, lm0, lm1, g0, g1,
                si, sg, ss):
    c = lax.axis_index("c")
    s = lax.axis_index("s")
    ibs = (ibs0, ibs1)
    ibd = (ibd0, ibd1)
    lms = (lm0, lm1)
    grows = (g0, g1)

    _zero2d(g0, 98)
    for t in range(16):
        pltpu.sync_copy(g0.at[pl.ds(0, 98)], acc_me.at[pl.ds(s * 1568 + t * 98, 98)])
    pltpu.sync_copy(g0.at[pl.ds(0, 98)], acc_em.at[pl.ds(s * 128, 98)])
    pltpu.sync_copy(g0.at[pl.ds(0, 30)], acc_em.at[pl.ds(s * 128 + 98, 30)])
    plsc.subcore_barrier()

    _ring_gs(98, 4, s * 50176, src, dst, bme, False, True, c * N_OP_HALF,
             acc_me, ibs, ibd, lms, grows, si, sg, ss)
    _ring_gs(98, 2, c * 401408 + s * 25088, src, dst, xop, True, False, None,
             acc_em, ibs, ibd, lms, grows, si, sg, ss)
    plsc.subcore_barrier()

    start = jnp.minimum(s * 1568, N_OP_HALF - 1568)
    for t in range(16):
        pltpu.sync_copy(acc_me.at[pl.ds(start + t * 98, 98)], g1.at[pl.ds(0, 98)])
        pltpu.sync_copy(g1.at[pl.ds(0, 98)],
                        me_out.at[pl.ds(c * N_OP_HALF + start + t * 98, 98)])
    plsc.subcore_barrier()
    _zero2d(g0, 98)
    for t in range(16):
        pltpu.sync_copy(g0.at[pl.ds(0, 98)], acc_me.at[pl.ds(s * 1568 + t * 98, 98)])
    plsc.subcore_barrier()

    _ring_gs(26, 4, s * 13312, psrc, pdst, xop, True, False, c * N_OP_HALF,
             acc_me, ibs, ibd, lms, grows, si, sg, ss)
    plsc.subcore_barrier()

    for t in range(16):
        pltpu.sync_copy(acc_me.at[pl.ds(start + t * 98, 98)], g1.at[pl.ds(0, 98)])
        pltpu.sync_copy(g1.at[pl.ds(0, 98)],
                        pp_out.at[pl.ds(c * N_OP_HALF + start + t * 98, 98)])
    pltpu.sync_copy(acc_em.at[pl.ds(s * 128, 128)], g0.at[pl.ds(0, 128)])
    pltpu.sync_copy(g0.at[pl.ds(0, 128)], em_out.at[c, pl.ds(s * 128, 128)])


_layer_call = pl.kernel(
    _layer_body,
    out_type=(
        jax.ShapeDtypeStruct((N_OP, HID), jnp.float32),
        jax.ShapeDtypeStruct((2, M_ACC, HID), jnp.float32),
        jax.ShapeDtypeStruct((N_OP, HID), jnp.float32),
    ),
    mesh=_MESH,
    compiler_params=_SC_PARAMS,
    scratch_types=[
        pltpu.VMEM_SHARED((OP_ACC, HID), jnp.float32),
        pltpu.VMEM_SHARED((M_ACC, HID), jnp.float32),
        pltpu.VMEM((512,), jnp.int32),
        pltpu.VMEM((512,), jnp.int32),
        pltpu.VMEM((512,), jnp.int32),
        pltpu.VMEM((512,), jnp.int32),
        pltpu.VMEM((CH,), jnp.int32),
        pltpu.VMEM((CH,), jnp.int32),
        pltpu.VMEM((CH, HID), jnp.float32),
        pltpu.VMEM((CH, HID), jnp.float32),
        pltpu.SemaphoreType.DMA,
        pltpu.SemaphoreType.DMA,
        pltpu.SemaphoreType.DMA,
    ],
)


def _pol_body(aop, am, ctm, p2s, iop, im, itm, lg_out,
              r10, r11, r20, r21, r30, r31, la0, la1,
              ibo0, ibo1, ibm0, ibm1, ibt0, ibt1, p2v, si, sg, sw):
    c = lax.axis_index("c")
    s = lax.axis_index("s")
    pltpu.sync_copy(p2s, p2v)
    r1 = (r10, r11)
    r2 = (r20, r21)
    r3 = (r30, r31)
    la = (la0, la1)
    ibo = (ibo0, ibo1)
    ibm = (ibm0, ibm1)
    ibt = (ibt0, ibt1)
    t0 = (c * 16 + s) * 25088
    nb = 98
    blk = 256

    def issue_i(bb, t):
        pltpu.async_copy(iop.at[pl.ds(t0 + bb * blk, blk)], ibo[t], si)
        pltpu.async_copy(im.at[pl.ds(t0 + bb * blk, blk)], ibm[t], si)
        pltpu.async_copy(itm.at[pl.ds(t0 + bb * blk, blk)], ibt[t], si)

    def wait_i(bb, t):
        for _ in range(3):
            pltpu.make_async_copy(iop.at[pl.ds(t0 + bb * blk, blk)], ibo[t], si).wait()

    def issue_g(t, j, cur):
        pltpu.async_copy(aop.at[ibo[t].at[pl.ds(j * CH, CH)]], r1[cur], sg)
        pltpu.async_copy(am.at[ibm[t].at[pl.ds(j * CH, CH)]], r2[cur], sg)
        pltpu.async_copy(ctm.at[ibt[t].at[pl.ds(j * CH, CH)]], r3[cur], sg)

    def wait_g(cur):
        for _ in range(3):
            pltpu.make_async_copy(aop.at[ibo[0].at[pl.ds(0, CH)]], r1[cur], sg).wait()

    def issue_w(k, prv):
        pltpu.async_copy(la[prv], lg_out.at[pl.ds(t0 + k * CH, CH)], sw)

    def wait_w(cur):
        pltpu.make_async_copy(la[cur], lg_out.at[pl.ds(t0, CH)], sw).wait()

    def compute(prv):
        def g_body(g16, _):
            rows = lax.iota(jnp.int32, 16) + g16 * 16

            acc = jnp.zeros((16,), jnp.float32)
            for j in range(HID):
                cols = jnp.full((16,), j, jnp.int32)
                h = (plsc.load_gather(r1[prv], [rows, cols])
                     + plsc.load_gather(r2[prv], [rows, cols])
                     + plsc.load_gather(r3[prv], [rows, cols]))
                acc = acc + jnp.maximum(h, 0.0) * p2v[pl.ds(j * 16, 16)]
            la[prv][pl.ds(g16 * 16, 16)] = acc
            return 0
        lax.fori_loop(0, CH // 16, g_body, 0)

    def pair_body(kk, _):
        for t in range(2):
            bb = 2 * kk + t
            wait_i(bb, t)
            for j in range(2):
                k = bb * 2 + j
                cur = j
                prv = 1 - j

                @pl.when(k >= 2)
                def _():
                    wait_w(cur)
                issue_g(t, j, cur)

                @pl.when(k >= 1)
                def _():
                    wait_g(prv)
                    compute(prv)
                    issue_w(k - 1, prv)

            @pl.when(bb + 1 < nb)
            def _():
                issue_i(bb + 1, 1 - t)
        return 0

    issue_i(0, 0)
    lax.fori_loop(0, nb // 2, pair_body, 0)
    wait_g(1)
    compute(1)
    issue_w(nb * 2 - 1, 1)
    wait_w(0)
    wait_w(1)


_pol_call = pl.kernel(
    _pol_body,
    out_type=jax.ShapeDtypeStruct((N_ACT_PAD,), jnp.float32),
    mesh=_MESH,
    compiler_params=_SC_PARAMS_NL,
    scratch_types=[
        pltpu.VMEM((CH, HID), jnp.float32),
        pltpu.VMEM((CH, HID), jnp.float32),
        pltpu.VMEM((CH, HID), jnp.float32),
        pltpu.VMEM((CH, HID), jnp.float32),
        pltpu.VMEM((CH, HID), jnp.float32),
        pltpu.VMEM((CH, HID), jnp.float32),
        pltpu.VMEM((CH,), jnp.float32),
        pltpu.VMEM((CH,), jnp.float32),
        pltpu.VMEM((256,), jnp.int32),
        pltpu.VMEM((256,), jnp.int32),
        pltpu.VMEM((256,), jnp.int32),
        pltpu.VMEM((256,), jnp.int32),
        pltpu.VMEM((256,), jnp.int32),
        pltpu.VMEM((256,), jnp.int32),
        pltpu.VMEM((HID * 16,), jnp.float32),
        pltpu.SemaphoreType.DMA,
        pltpu.SemaphoreType.DMA,
        pltpu.SemaphoreType.DMA,
    ],
)


def _softmax_body(x_ref, o_ref):
    x = x_ref[...]
    m = jnp.max(x)
    e = jnp.exp(x - m)
    o_ref[...] = e / jnp.sum(e)


def _pallas_softmax(logits):
    n = logits.shape[0]
    cols = 1024
    rows = (n + cols - 1) // cols
    pad = rows * cols - n
    x = jnp.concatenate([logits, jnp.full((pad,), -1e30, logits.dtype)]).reshape(rows, cols)
    y = pl.pallas_call(
        _softmax_body,
        out_shape=jax.ShapeDtypeStruct((rows, cols), jnp.float32),
    )(x)
    return y.reshape(-1)[:n]


def _ln(x, g, b):
    m = x.mean(-1, keepdims=True)
    v = ((x - m) ** 2).mean(-1, keepdims=True)
    return (x - m) / jnp.sqrt(v + 1e-5) * g + b


def _pad_rows(x, n):
    return jnp.concatenate([x, jnp.zeros((n - x.shape[0],) + x.shape[1:], x.dtype)])


def _pad_idx(x, n, fill):
    return jnp.concatenate([x.astype(jnp.int32),
                            jnp.full((n - x.shape[0],), fill, jnp.int32)])


def kernel(op_x, machine_x, global_features, params, eligible_edge_index,
           precedes_edge_index, legal_mask, action_op_idx, action_machine_idx, action_tm_idx):
    p = params
    f32 = jnp.float32
    src = _pad_idx(eligible_edge_index[0], E_EL_PAD, N_OP)
    dst = _pad_idx(eligible_edge_index[1], E_EL_PAD, N_M)
    psrc = _pad_idx(precedes_edge_index[0], E_PR_PAD, N_OP)
    pdst = _pad_idx(precedes_edge_index[1], E_PR_PAD, N_OP)
    ones_h = jnp.ones((512,), f32)

    dme2, dm2, dpp2 = _deg_call(src, dst, pdst, ones_h)
    inv_me = 1.0 / jnp.clip(dme2[:N_OP] + dme2[D_ACC:D_ACC + N_OP], 1.0)[:, None]
    inv_m = 1.0 / jnp.clip(dm2[:N_M] + dm2[M_ACC:M_ACC + N_M], 1.0)[:, None]
    inv_pp = 1.0 / jnp.clip(dpp2[:N_OP] + dpp2[D_ACC:D_ACC + N_OP], 1.0)[:, None]

    x_op = jax.nn.relu(op_x @ p['op_W'].T + p['op_b'])
    x_m = jax.nn.relu(machine_x @ p['m_W'].T + p['m_b'])
    for i in range(NL):
        si = str(i)
        bme = _pad_rows(x_m @ p['me' + si + '_Wl'].T, M_ACC)
        xopp = _pad_rows(x_op, D_ACC)
        me_sum, em_part, pp_sum = _layer_call(xopp, bme, src, dst, psrc, pdst)
        em_sum = em_part[0, :N_M] + em_part[1, :N_M]
        out_m = (em_sum * inv_m) @ p['em' + si + '_Wl'].T + p['em' + si + '_bl'] \
            + x_m @ p['em' + si + '_Wr'].T
        out_op = me_sum * inv_me + p['me' + si + '_bl'] \
            + (pp_sum * inv_pp) @ p['pp' + si + '_Wl'].T + p['pp' + si + '_bl'] \
            + x_op @ (p['me' + si + '_Wr'] + p['pp' + si + '_Wr']).T
        x_op = _ln(x_op + out_op, p['lnop' + si + '_g'], p['lnop' + si + '_b'])
        x_m = _ln(x_m + out_m, p['lnm' + si + '_g'], p['lnm' + si + '_b'])

    P1 = p['P1']
    aop_t = _pad_rows(x_op @ P1[:, :HID].T, D_ACC)
    am_t = _pad_rows(x_m @ P1[:, HID:2 * HID].T, M_ACC)
    ctm = p['tm_features'] @ P1[:, 2 * HID:].T + p['pb1']
    p2s = jnp.broadcast_to(p['P2'][0][:, None], (HID, 16)).reshape(-1)
    iop = _pad_idx(action_op_idx, N_ACT_PAD, 0)
    im = _pad_idx(action_machine_idx, N_ACT_PAD, 0)
    itm = _pad_idx(action_tm_idx, N_ACT_PAD, 0)
    action_logits = _pol_call(aop_t, am_t, ctm, p2s, iop, im, itm)[
        :action_op_idx.shape[0]] + p['pb2'][0]
    hi = jax.nn.relu(p['idle'] @ P1.T + p['pb1'])
    idle_logit = hi @ p['P2'][0] + p['pb2'][0]
    all_logits = jnp.concatenate([action_logits, idle_logit[None]])
    all_logits = jnp.where(legal_mask, all_logits, -1e9)
    policy = _pallas_softmax(all_logits)
    v_in = jnp.concatenate([x_op.mean(axis=0), x_m.mean(axis=0), global_features[0]])
    hv = jax.nn.relu(v_in @ p['V1'].T + p['vb1'])
    value = jnp.tanh((hv @ p['V2'].T + p['vb2'])[0])
    return policy, value


# speedup vs baseline: 1.6337x; 1.6337x over previous
import jax
import jax.numpy as jnp
from jax import lax
from jax.experimental import pallas as pl
from jax.experimental.pallas import tpu as pltpu
from jax.experimental.pallas import tpu_sc as plsc

N_OP = 50000
N_M = 2000
N_TM = 8
HID = 64
NL = 3

E_EL_PAD = 802816
E_PR_PAD = 212992
N_ACT_PAD = 802816

N_OP_HALF = 25000
OP_ACC = 25088
M_ACC = 2048
D_ACC = 50048

CH = 128

_MESH = plsc.VectorSubcoreMesh(core_axis_name="c", subcore_axis_name="s", num_cores=2, num_subcores=16)
_SC_PARAMS = pltpu.CompilerParams(use_tc_tiling_on_sc=False)
_SC_PARAMS_NL = pltpu.CompilerParams(use_tc_tiling_on_sc=False,
                                     needs_layout_passes=False)


def _zero2d(buf, rows):
    def body(i, _):
        for j in range(4):
            buf[i, pl.ds(j * 16, 16)] = jnp.zeros((16,), jnp.float32)
        return 0
    lax.fori_loop(0, rows, body, 0)


def _zero1d(buf, n):
    def body(i, _):
        buf[pl.ds(i * 16, 16)] = jnp.zeros((16,), jnp.float32)
        return 0
    lax.fori_loop(0, n // 16, body, 0)


def _addrows(dst_ref, src_ref, n):
    def body(i, _):
        for j in range(4):
            sl = pl.ds(j * 16, 16)
            dst_ref[i, sl] = dst_ref[i, sl] + src_ref[i, sl]
        return 0
    lax.fori_loop(0, n, body, 0)


def _ring_gs(nb, bpc, t0, src_s, src_d, tab, gather_by_s, scat_by_s,
             clamp_base, acc, ibs, ibd, lms, grows, si, sg, ss):
    blk = bpc * CH
    n = nb * bpc

    def issue_i(bb, t):
        pltpu.async_copy(src_s.at[pl.ds(t0 + bb * blk, blk)], ibs[t].at[pl.ds(0, blk)], si)
        pltpu.async_copy(src_d.at[pl.ds(t0 + bb * blk, blk)], ibd[t].at[pl.ds(0, blk)], si)

    def wait_i(bb, t):
        pltpu.make_async_copy(src_s.at[pl.ds(t0 + bb * blk, blk)], ibs[t].at[pl.ds(0, blk)], si).wait()
        pltpu.make_async_copy(src_d.at[pl.ds(t0 + bb * blk, blk)], ibd[t].at[pl.ds(0, blk)], si).wait()

    def issue_g(t, j, cur):
        ib = ibs[t] if gather_by_s else ibd[t]
        pltpu.async_copy(tab.at[ib.at[pl.ds(j * CH, CH)]], grows[cur], sg)

    def wait_g(cur):
        ib = ibs[0] if gather_by_s else ibd[0]
        pltpu.make_async_copy(tab.at[ib.at[pl.ds(0, CH)]], grows[cur], sg).wait()

    def issue_s(prv):
        pltpu.async_copy(grows[prv], acc.at[lms[prv]], ss, add=True)

    def wait_s(prv):
        pltpu.make_async_copy(grows[prv], acc.at[lms[prv]], ss).wait()

    def lm_compute(t, j, cur):
        ib = ibs[t] if scat_by_s else ibd[t]
        for q in range(CH // 16):
            v = ib[pl.ds(j * CH + q * 16, 16)]
            if clamp_base is not None:
                l = v - clamp_base
                ok = (l >= 0) & (l < N_OP_HALF)
                lms[cur][pl.ds(q * 16, 16)] = jnp.where(ok, l, N_OP_HALF)
            else:
                lms[cur][pl.ds(q * 16, 16)] = v

    def pair_body(kk, _):
        for t in range(2):
            bb = 2 * kk + t
            wait_i(bb, t)
            for j in range(bpc):
                k = bb * bpc + j
                cur = j % 2
                prv = 1 - cur
                @pl.when(k >= 2)
                def _():
                    wait_s(cur)
                lm_compute(t, j, cur)
                issue_g(t, j, cur)

                @pl.when(k >= 1)
                def _():
                    wait_g(prv)
                    issue_s(prv)

            @pl.when(bb + 1 < nb)
            def _():
                issue_i(bb + 1, 1 - t)
        return 0

    issue_i(0, 0)
    lax.fori_loop(0, nb // 2, pair_body, 0)
    wait_g(1)
    issue_s(1)
    wait_s(0)
    wait_s(1)


def _deg_body(src, dst, ppd, ones_h, dme, dm, dpp,
              acc_me, acc_m, acc_pp, idx, ones_v, zbuf):
    c = lax.axis_index("c")
    s = lax.axis_index("s")
    _zero1d(zbuf, 3128)
    pltpu.sync_copy(zbuf, acc_me.at[pl.ds(s * 3128, 3128)])
    pltpu.sync_copy(zbuf, acc_pp.at[pl.ds(s * 3128, 3128)])
    pltpu.sync_copy(zbuf.at[pl.ds(0, 128)], acc_m.at[pl.ds(s * 128, 128)])
    pltpu.sync_copy(ones_h, ones_v)
    plsc.subcore_barrier()

    def el_chunk(k, _):
        b = c * 401408 + s * 25088 + k * 512
        pltpu.sync_copy(src.at[pl.ds(b, 512)], idx)
        pltpu.sync_copy(ones_v, acc_me.at[idx], add=True)
        pltpu.sync_copy(dst.at[pl.ds(b, 512)], idx)
        pltpu.sync_copy(ones_v, acc_m.at[idx], add=True)
        return 0
    lax.fori_loop(0, 49, el_chunk, 0)

    def pp_chunk(k, _):
        b = c * 106496 + s * 6656 + k * 512
        pltpu.sync_copy(ppd.at[pl.ds(b, 512)], idx)
        pltpu.sync_copy(ones_v, acc_pp.at[idx], add=True)
        return 0
    lax.fori_loop(0, 13, pp_chunk, 0)
    plsc.subcore_barrier()
    pltpu.sync_copy(acc_me.at[pl.ds(s * 3128, 3128)], zbuf)
    pltpu.sync_copy(zbuf, dme.at[pl.ds(c * D_ACC + s * 3128, 3128)])
    pltpu.sync_copy(acc_pp.at[pl.ds(s * 3128, 3128)], zbuf)
    pltpu.sync_copy(zbuf, dpp.at[pl.ds(c * D_ACC + s * 3128, 3128)])
    pltpu.sync_copy(acc_m.at[pl.ds(s * 128, 128)], zbuf.at[pl.ds(0, 128)])
    pltpu.sync_copy(zbuf.at[pl.ds(0, 128)], dm.at[pl.ds(c * M_ACC + s * 128, 128)])


_deg_call = pl.kernel(
    _deg_body,
    out_type=(
        jax.ShapeDtypeStruct((2 * D_ACC,), jnp.float32),
        jax.ShapeDtypeStruct((2 * M_ACC,), jnp.float32),
        jax.ShapeDtypeStruct((2 * D_ACC,), jnp.float32),
    ),
    mesh=_MESH,
    compiler_params=_SC_PARAMS,
    scratch_types=[
        pltpu.VMEM_SHARED((D_ACC,), jnp.float32),
        pltpu.VMEM_SHARED((M_ACC,), jnp.float32),
        pltpu.VMEM_SHARED((D_ACC,), jnp.float32),
        pltpu.VMEM((512,), jnp.int32),
        pltpu.VMEM((512,), jnp.float32),
        pltpu.VMEM((3128,), jnp.float32),
    ],
)


def _layer_body(xop, bme, src, dst, psrc, pdst, me_out, em_out, pp_out,
                acc_me, acc_em, ibs0, ibs1, ibd0, ibd1, lm0, lm1, g0, g1,
                si, sg, ss):
    c = lax.axis_index("c")
    s = lax.axis_index("s")
    ibs = (ibs0, ibs1)
    ibd = (ibd0, ibd1)
    lms = (lm0, lm1)
    grows = (g0, g1)

    _zero2d(g0, 98)
    for t in range(16):
        pltpu.sync_copy(g0.at[pl.ds(0, 98)], acc_me.at[pl.ds(s * 1568 + t * 98, 98)])
    pltpu.sync_copy(g0.at[pl.ds(0, 98)], acc_em.at[pl.ds(s * 128, 98)])
    pltpu.sync_copy(g0.at[pl.ds(0, 30)], acc_em.at[pl.ds(s * 128 + 98, 30)])
    plsc.subcore_barrier()

    _ring_gs(98, 4, s * 50176, src, dst, bme, False, True, c * N_OP_HALF,
             acc_me, ibs, ibd, lms, grows, si, sg, ss)
    _ring_gs(98, 2, c * 401408 + s * 25088, src, dst, xop, True, False, None,
             acc_em, ibs, ibd, lms, grows, si, sg, ss)
    plsc.subcore_barrier()

    start = jnp.minimum(s * 1568, N_OP_HALF - 1568)
    for t in range(16):
        pltpu.sync_copy(acc_me.at[pl.ds(start + t * 98, 98)], g1.at[pl.ds(0, 98)])
        pltpu.sync_copy(g1.at[pl.ds(0, 98)],
                        me_out.at[pl.ds(c * N_OP_HALF + start + t * 98, 98)])
    plsc.subcore_barrier()
    _zero2d(g0, 98)
    for t in range(16):
        pltpu.sync_copy(g0.at[pl.ds(0, 98)], acc_me.at[pl.ds(s * 1568 + t * 98, 98)])
    plsc.subcore_barrier()

    _ring_gs(26, 4, s * 13312, psrc, pdst, xop, True, False, c * N_OP_HALF,
             acc_me, ibs, ibd, lms, grows, si, sg, ss)
    plsc.subcore_barrier()

    for t in range(16):
        pltpu.sync_copy(acc_me.at[pl.ds(start + t * 98, 98)], g1.at[pl.ds(0, 98)])
        pltpu.sync_copy(g1.at[pl.ds(0, 98)],
                        pp_out.at[pl.ds(c * N_OP_HALF + start + t * 98, 98)])
    pltpu.sync_copy(acc_em.at[pl.ds(s * 128, 128)], g0.at[pl.ds(0, 128)])
    pltpu.sync_copy(g0.at[pl.ds(0, 128)], em_out.at[c, pl.ds(s * 128, 128)])


_layer_call = pl.kernel(
    _layer_body,
    out_type=(
        jax.ShapeDtypeStruct((N_OP, HID), jnp.float32),
        jax.ShapeDtypeStruct((2, M_ACC, HID), jnp.float32),
        jax.ShapeDtypeStruct((N_OP, HID), jnp.float32),
    ),
    mesh=_MESH,
    compiler_params=_SC_PARAMS,
    scratch_types=[
        pltpu.VMEM_SHARED((OP_ACC, HID), jnp.float32),
        pltpu.VMEM_SHARED((M_ACC, HID), jnp.float32),
        pltpu.VMEM((512,), jnp.int32),
        pltpu.VMEM((512,), jnp.int32),
        pltpu.VMEM((512,), jnp.int32),
        pltpu.VMEM((512,), jnp.int32),
        pltpu.VMEM((CH,), jnp.int32),
        pltpu.VMEM((CH,), jnp.int32),
        pltpu.VMEM((CH, HID), jnp.float32),
        pltpu.VMEM((CH, HID), jnp.float32),
        pltpu.SemaphoreType.DMA,
        pltpu.SemaphoreType.DMA,
        pltpu.SemaphoreType.DMA,
    ],
)


def _pol_body(aop, amt, iop, iam, h_out,
              r10, r11, r20, r21, ibo0, ibo1, ibm0, ibm1, si, sg, sw):
    c = lax.axis_index("c")
    s = lax.axis_index("s")
    r1 = (r10, r11)
    r2 = (r20, r21)
    ibo = (ibo0, ibo1)
    ibm = (ibm0, ibm1)
    t0 = (c * 16 + s) * 25088
    nb = 98
    blk = 256

    def issue_i(bb, t):
        pltpu.async_copy(iop.at[pl.ds(t0 + bb * blk, blk)], ibo[t], si)
        pltpu.async_copy(iam.at[pl.ds(t0 + bb * blk, blk)], ibm[t], si)

    def wait_i(bb, t):
        for _ in range(2):
            pltpu.make_async_copy(iop.at[pl.ds(t0 + bb * blk, blk)], ibo[t], si).wait()

    def issue_g(t, j, cur):
        pltpu.async_copy(aop.at[ibo[t].at[pl.ds(j * CH, CH)]], r1[cur], sg)
        pltpu.async_copy(amt.at[ibm[t].at[pl.ds(j * CH, CH)]], r2[cur], sg)

    def wait_g(cur):
        for _ in range(2):
            pltpu.make_async_copy(aop.at[ibo[0].at[pl.ds(0, CH)]], r1[cur], sg).wait()

    def issue_w(k, prv):
        pltpu.async_copy(r1[prv], h_out.at[pl.ds(t0 + k * CH, CH)], sw)

    def wait_w(cur):
        pltpu.make_async_copy(r1[cur], h_out.at[pl.ds(t0, CH)], sw).wait()

    def compute(prv):
        def a_body(i, _):
            for q in range(4):
                sl = pl.ds(q * 16, 16)
                r1[prv][i, sl] = jnp.maximum(r1[prv][i, sl] + r2[prv][i, sl], 0.0)
            return 0
        lax.fori_loop(0, CH, a_body, 0)

    def pair_body(kk, _):
        for t in range(2):
            bb = 2 * kk + t
            wait_i(bb, t)
            for j in range(2):
                k = bb * 2 + j
                cur = j
                prv = 1 - j

                @pl.when(k >= 2)
                def _():
                    wait_w(cur)
                issue_g(t, j, cur)

                @pl.when(k >= 1)
                def _():
                    wait_g(prv)
                    compute(prv)
                    issue_w(k - 1, prv)

            @pl.when(bb + 1 < nb)
            def _():
                issue_i(bb + 1, 1 - t)
        return 0

    issue_i(0, 0)
    lax.fori_loop(0, nb // 2, pair_body, 0)
    wait_g(1)
    compute(1)
    issue_w(nb * 2 - 1, 1)
    wait_w(0)
    wait_w(1)


_pol_call = pl.kernel(
    _pol_body,
    out_type=jax.ShapeDtypeStruct((N_ACT_PAD, HID), jnp.float32),
    mesh=_MESH,
    compiler_params=_SC_PARAMS,
    scratch_types=[
        pltpu.VMEM((CH, HID), jnp.float32),
        pltpu.VMEM((CH, HID), jnp.float32),
        pltpu.VMEM((CH, HID), jnp.float32),
        pltpu.VMEM((CH, HID), jnp.float32),
        pltpu.VMEM((256,), jnp.int32),
        pltpu.VMEM((256,), jnp.int32),
        pltpu.VMEM((256,), jnp.int32),
        pltpu.VMEM((256,), jnp.int32),
        pltpu.SemaphoreType.DMA,
        pltpu.SemaphoreType.DMA,
        pltpu.SemaphoreType.DMA,
    ],
)


def _softmax_body(x_ref, o_ref):
    x = x_ref[...]
    m = jnp.max(x)
    e = jnp.exp(x - m)
    o_ref[...] = e / jnp.sum(e)


def _pallas_softmax(logits):
    n = logits.shape[0]
    cols = 1024
    rows = (n + cols - 1) // cols
    pad = rows * cols - n
    x = jnp.concatenate([logits, jnp.full((pad,), -1e30, logits.dtype)]).reshape(rows, cols)
    y = pl.pallas_call(
        _softmax_body,
        out_shape=jax.ShapeDtypeStruct((rows, cols), jnp.float32),
    )(x)
    return y.reshape(-1)[:n]


def _ln(x, g, b):
    m = x.mean(-1, keepdims=True)
    v = ((x - m) ** 2).mean(-1, keepdims=True)
    return (x - m) / jnp.sqrt(v + 1e-5) * g + b


def _pad_rows(x, n):
    return jnp.concatenate([x, jnp.zeros((n - x.shape[0],) + x.shape[1:], x.dtype)])


def _pad_idx(x, n, fill):
    return jnp.concatenate([x.astype(jnp.int32),
                            jnp.full((n - x.shape[0],), fill, jnp.int32)])


def kernel(op_x, machine_x, global_features, params, eligible_edge_index,
           precedes_edge_index, legal_mask, action_op_idx, action_machine_idx, action_tm_idx):
    p = params
    f32 = jnp.float32
    src = _pad_idx(eligible_edge_index[0], E_EL_PAD, N_OP)
    dst = _pad_idx(eligible_edge_index[1], E_EL_PAD, N_M)
    psrc = _pad_idx(precedes_edge_index[0], E_PR_PAD, N_OP)
    pdst = _pad_idx(precedes_edge_index[1], E_PR_PAD, N_OP)
    ones_h = jnp.ones((512,), f32)

    dme2, dm2, dpp2 = _deg_call(src, dst, pdst, ones_h)
    inv_me = 1.0 / jnp.clip(dme2[:N_OP] + dme2[D_ACC:D_ACC + N_OP], 1.0)[:, None]
    inv_m = 1.0 / jnp.clip(dm2[:N_M] + dm2[M_ACC:M_ACC + N_M], 1.0)[:, None]
    inv_pp = 1.0 / jnp.clip(dpp2[:N_OP] + dpp2[D_ACC:D_ACC + N_OP], 1.0)[:, None]

    x_op = jax.nn.relu(op_x @ p['op_W'].T + p['op_b'])
    x_m = jax.nn.relu(machine_x @ p['m_W'].T + p['m_b'])
    for i in range(NL):
        si = str(i)
        bme = _pad_rows(x_m @ p['me' + si + '_Wl'].T, M_ACC)
        xopp = _pad_rows(x_op, D_ACC)
        me_sum, em_part, pp_sum = _layer_call(xopp, bme, src, dst, psrc, pdst)
        em_sum = em_part[0, :N_M] + em_part[1, :N_M]
        out_m = (em_sum * inv_m) @ p['em' + si + '_Wl'].T + p['em' + si + '_bl'] \
            + x_m @ p['em' + si + '_Wr'].T
        out_op = me_sum * inv_me + p['me' + si + '_bl'] \
            + (pp_sum * inv_pp) @ p['pp' + si + '_Wl'].T + p['pp' + si + '_bl'] \
            + x_op @ (p['me' + si + '_Wr'] + p['pp' + si + '_Wr']).T
        x_op = _ln(x_op + out_op, p['lnop' + si + '_g'], p['lnop' + si + '_b'])
        x_m = _ln(x_m + out_m, p['lnm' + si + '_g'], p['lnm' + si + '_b'])

    P1 = p['P1']
    aop_t = _pad_rows(x_op @ P1[:, :HID].T, D_ACC)
    ctm = p['tm_features'] @ P1[:, 2 * HID:].T + p['pb1']
    amt_t = _pad_rows(((x_m @ P1[:, HID:2 * HID].T)[:, None, :]
                       + ctm[None, :, :]).reshape(N_M * N_TM, HID), 16384)
    iop = _pad_idx(action_op_idx, N_ACT_PAD, 0)
    iamt = _pad_idx(action_machine_idx * N_TM + action_tm_idx, N_ACT_PAD, 0)
    h_all = _pol_call(aop_t, amt_t, iop, iamt)[:action_op_idx.shape[0]]
    action_logits = h_all @ p['P2'][0] + p['pb2'][0]
    hi = jax.nn.relu(p['idle'] @ P1.T + p['pb1'])
    idle_logit = hi @ p['P2'][0] + p['pb2'][0]
    all_logits = jnp.concatenate([action_logits, idle_logit[None]])
    all_logits = jnp.where(legal_mask, all_logits, -1e9)
    policy = _pallas_softmax(all_logits)
    v_in = jnp.concatenate([x_op.mean(axis=0), x_m.mean(axis=0), global_features[0]])
    hv = jax.nn.relu(v_in @ p['V1'].T + p['vb1'])
    value = jnp.tanh((hv @ p['V2'].T + p['vb2'])[0])
    return policy, value
